# baseline (device time: 87626 ns/iter reference)
import os

import jax
import jax.numpy as jnp
from jax import lax
from jax.experimental import pallas as pl
from jax.experimental.pallas import tpu as pltpu

N_DEV = 4
B, SQ, DM = 4, 256, 1024
HG, HL, DH = 32, 8, 128
NQB, QBLK = 4, 64
NT = 16
KSEL = NT * QBLK
HALF = SQ // 2
SCALE = 0.08838834764831843
ABLATE = os.environ.get("ABLATE", "")
_SKIP_DMA = ABLATE == "compute"
_SKIP_MATH = ABLATE == "mem"
OFF = (0, 1, 3, 2)

A1R, A1L, A2R, A2L, T1R, T1L, T2R, T2L, T3R, T3L = range(10)


def kernel(x, Wq, K_ext, V_ext, Wo):
    K_r = K_ext.reshape(B, NT, NQB, QBLK, HG, DH)
    V_r = V_ext.reshape(B, NT, NQB, QBLK, HG, DH)

    def body(x_ref, wq_ref, k_hbm, v_hbm, wo_ref, out_ref,
             x_all, wq_bf, wo_bf, q_bf, kst, vst, k_bf, v_bf, ctx_bf,
             acc_my, rs_out_r, rs_out_l, rs_diag, rs_in_a, rs_in_b,
             relay_r, relay_l, diag_a, diag_b,
             snd, rcv, k_sems, v_sems):
        my = lax.axis_index("i")
        right = lax.rem(my + 1, N_DEV)
        left = lax.rem(my + N_DEV - 1, N_DEV)
        h0 = my * HL

        def batch_of(j):
            return lax.rem(my + OFF[j], N_DEV)

        def remote(src, dst, i, dev):
            r = pltpu.make_async_remote_copy(
                src_ref=src, dst_ref=dst, send_sem=snd.at[i],
                recv_sem=rcv.at[i], device_id=(dev,),
                device_id_type=pl.DeviceIdType.MESH)
            r.start()
            return r

        barrier = pltpu.get_barrier_semaphore()
        for nbr in (left, right):
            pl.semaphore_signal(barrier, inc=1, device_id=(nbr,),
                                device_id_type=pl.DeviceIdType.MESH)
        pl.semaphore_wait(barrier, 2)

        x_all[pl.ds(my, 1)] = x_ref[...].astype(jnp.bfloat16)
        a1r = remote(x_all.at[pl.ds(my, 1)], x_all.at[pl.ds(my, 1)],
                     A1R, right)
        a1l = remote(x_all.at[pl.ds(my, 1)], x_all.at[pl.ds(my, 1)],
                     A1L, left)

        def issue_stage(t):
            j, qb = divmod(t, NQB)
            bb = batch_of(j)
            slot = t % 2
            ck = pltpu.make_async_copy(
                k_hbm.at[bb, :, qb, :, pl.ds(h0, HL), :], kst.at[slot],
                k_sems.at[slot])
            cv = pltpu.make_async_copy(
                v_hbm.at[bb, :, qb, :, pl.ds(h0, HL), :], vst.at[slot],
                v_sems.at[slot])
            ck.start()
            cv.start()
            return (ck, cv)

        desc = {} if _SKIP_DMA else {0: issue_stage(0), 1: issue_stage(1)}

        wq_bf[...] = wq_ref[...].astype(jnp.bfloat16)
        wo_bf[...] = wo_ref[...].astype(jnp.bfloat16)

        def compute_batch(j, store):
            bb = batch_of(j)
            xb = x_all[pl.ds(bb, 1)][0]
            q = jnp.dot(xb, wq_bf[...],
                        preferred_element_type=jnp.float32)
            q_bf[...] = (q * SCALE).astype(jnp.bfloat16)

            for qb in range(NQB):
                t = j * NQB + qb
                slot = t % 2
                if not _SKIP_DMA:
                    ck, cv = desc.pop(t)
                    ck.wait()
                    cv.wait()
                    k_bf[...] = kst[slot].reshape(KSEL, HL * DH
                                                  ).astype(jnp.bfloat16)
                    v_bf[...] = vst[slot].reshape(KSEL, HL * DH
                                                  ).astype(jnp.bfloat16)
                    if t + 2 < N_DEV * NQB:
                        desc[t + 2] = issue_stage(t + 2)

                for h in range(HL) if not _SKIP_MATH else ():
                    kh = k_bf[:, h * DH:(h + 1) * DH]
                    vh = v_bf[:, h * DH:(h + 1) * DH]
                    qh = q_bf[qb * QBLK:(qb + 1) * QBLK,
                              h * DH:(h + 1) * DH]
                    s = lax.dot_general(qh, kh, (((1,), (1,)), ((), ())),
                                        preferred_element_type=jnp.float32)
                    e = jnp.exp(s)
                    inv = 1.0 / jnp.sum(e, axis=-1, keepdims=True)
                    o = jnp.dot(e.astype(jnp.bfloat16), vh,
                                preferred_element_type=jnp.float32)
                    ctx_bf[:, h * DH:(h + 1) * DH] = (o * inv
                                                      ).astype(jnp.bfloat16)
                psum = jnp.dot(ctx_bf[...], wo_bf[...],
                               preferred_element_type=jnp.float32)
                store(qb, psum)

        def store_f32(ref):
            def f(qb, psum):
                ref[0, qb * QBLK:(qb + 1) * QBLK, :] = psum
            return f

        def store_bf16(ref):
            def f(qb, psum):
                ref[0, qb * QBLK:(qb + 1) * QBLK, :] = psum.astype(
                    jnp.bfloat16)
            return f

        compute_batch(0, store_f32(acc_my))

        a1r.wait_recv()
        a1l.wait_recv()
        a2r = remote(x_all.at[pl.ds(left, 1), pl.ds(0, HALF), :],
                     x_all.at[pl.ds(left, 1), pl.ds(0, HALF), :],
                     A2R, right)
        a2l = remote(x_all.at[pl.ds(right, 1), pl.ds(HALF, HALF), :],
                     x_all.at[pl.ds(right, 1), pl.ds(HALF, HALF), :],
                     A2L, left)

        compute_batch(1, store_bf16(rs_out_r))
        t1r = remote(rs_out_r.at[...], rs_in_a.at[...], T1R, right)

        compute_batch(2, store_bf16(rs_out_l))
        t1l = remote(rs_out_l.at[...], rs_in_b.at[...], T1L, left)

        a2r.wait_recv()
        a2l.wait_recv()
        compute_batch(3, store_bf16(rs_diag))

        t2r = remote(rs_diag.at[:, pl.ds(0, HALF), :], relay_r.at[...],
                     T2R, right)
        t2l = remote(rs_diag.at[:, pl.ds(HALF, HALF), :], relay_l.at[...],
                     T2L, left)

        t1r.wait_recv()
        t1l.wait_recv()
        base = (acc_my[...] + rs_in_a[...].astype(jnp.float32)
                + rs_in_b[...].astype(jnp.float32))

        t2r.wait_recv()
        t3r = remote(relay_r.at[...], diag_a.at[...], T3R, right)
        t2l.wait_recv()
        t3l = remote(relay_l.at[...], diag_b.at[...], T3L, left)

        t3r.wait_recv()
        t3l.wait_recv()
        out_ref[:, 0:HALF, :] = (base[:, 0:HALF, :]
                                 + diag_a[...].astype(jnp.float32))
        out_ref[:, HALF:SQ, :] = (base[:, HALF:SQ, :]
                                  + diag_b[...].astype(jnp.float32))

        for r in (a1r, a1l, a2r, a2l, t1r, t1l, t2r, t2l, t3r, t3l):
            r.wait_send()

    return pl.pallas_call(
        body,
        out_shape=jax.ShapeDtypeStruct((1, SQ, DM), jnp.float32),
        in_specs=[
            pl.BlockSpec(memory_space=pltpu.MemorySpace.VMEM),
            pl.BlockSpec(memory_space=pltpu.MemorySpace.VMEM),
            pl.BlockSpec(memory_space=pl.ANY),
            pl.BlockSpec(memory_space=pl.ANY),
            pl.BlockSpec(memory_space=pltpu.MemorySpace.VMEM),
        ],
        out_specs=pl.BlockSpec(memory_space=pltpu.MemorySpace.VMEM),
        scratch_shapes=[
            pltpu.VMEM((B, SQ, DM), jnp.bfloat16),
            pltpu.VMEM((DM, DM), jnp.bfloat16),
            pltpu.VMEM((DM, DM), jnp.bfloat16),
            pltpu.VMEM((SQ, HL * DH), jnp.bfloat16),
            pltpu.VMEM((2, NT, QBLK, HL, DH), jnp.float32),
            pltpu.VMEM((2, NT, QBLK, HL, DH), jnp.float32),
            pltpu.VMEM((KSEL, HL * DH), jnp.bfloat16),
            pltpu.VMEM((KSEL, HL * DH), jnp.bfloat16),
            pltpu.VMEM((QBLK, HL * DH), jnp.bfloat16),
            pltpu.VMEM((1, SQ, DM), jnp.float32),
            pltpu.VMEM((1, SQ, DM), jnp.bfloat16),
            pltpu.VMEM((1, SQ, DM), jnp.bfloat16),
            pltpu.VMEM((1, SQ, DM), jnp.bfloat16),
            pltpu.VMEM((1, SQ, DM), jnp.bfloat16),
            pltpu.VMEM((1, SQ, DM), jnp.bfloat16),
            pltpu.VMEM((1, HALF, DM), jnp.bfloat16),
            pltpu.VMEM((1, HALF, DM), jnp.bfloat16),
            pltpu.VMEM((1, HALF, DM), jnp.bfloat16),
            pltpu.VMEM((1, HALF, DM), jnp.bfloat16),
            pltpu.SemaphoreType.DMA((10,)),
            pltpu.SemaphoreType.DMA((10,)),
            pltpu.SemaphoreType.DMA((2,)),
            pltpu.SemaphoreType.DMA((2,)),
        ],
        compiler_params=pltpu.CompilerParams(
            collective_id=0, vmem_limit_bytes=64 * 1024 * 1024),
    )(x, Wq, K_r, V_r, Wo)


# device time: 81924 ns/iter; 1.0696x vs baseline; 1.0696x over previous
import os

import jax
import jax.numpy as jnp
from jax import lax
from jax.experimental import pallas as pl
from jax.experimental.pallas import tpu as pltpu

N_DEV = 4
B, SQ, DM = 4, 256, 1024
HG, HL, DH = 32, 8, 128
NQB, QBLK = 4, 64
NT = 16
KSEL = NT * QBLK
HALF = SQ // 2
SCALE = 0.08838834764831843
ABLATE = os.environ.get("ABLATE", "")
_SKIP_DMA = ABLATE == "compute"
_SKIP_MATH = ABLATE == "mem"
OFF = (0, 1, 2, 3)

A1R, A1L, A2R, A2L, T1R, T1L, T2R, T2L, T3R, T3L = range(10)


def kernel(x, Wq, K_ext, V_ext, Wo):
    K_r = K_ext.reshape(B, NT, NQB, QBLK, HG, DH)
    V_r = V_ext.reshape(B, NT, NQB, QBLK, HG, DH)

    def body(x_ref, wq_ref, k_hbm, v_hbm, wo_ref, out_ref,
             x_all, wq_bf, wo_bf, q_bf, kst, vst, k_bf, v_bf, ctx_bf,
             scores, e_buf,
             acc_my, rs_out_r, rs_out_l, rs_diag, rs_in_a, rs_in_b,
             relay_r, relay_l, diag_a, diag_b,
             snd, rcv, k_sems, v_sems):
        my = lax.axis_index("i")
        right = lax.rem(my + 1, N_DEV)
        left = lax.rem(my + N_DEV - 1, N_DEV)
        h0 = my * HL

        def batch_of(j):
            return lax.rem(my + OFF[j], N_DEV)

        def remote(src, dst, i, dev):
            r = pltpu.make_async_remote_copy(
                src_ref=src, dst_ref=dst, send_sem=snd.at[i],
                recv_sem=rcv.at[i], device_id=(dev,),
                device_id_type=pl.DeviceIdType.MESH)
            r.start()
            return r

        barrier = pltpu.get_barrier_semaphore()
        for nbr in (left, right):
            pl.semaphore_signal(barrier, inc=1, device_id=(nbr,),
                                device_id_type=pl.DeviceIdType.MESH)
        pl.semaphore_wait(barrier, 2)

        x_all[pl.ds(my, 1)] = x_ref[...].astype(jnp.bfloat16)
        a1r = remote(x_all.at[pl.ds(my, 1)], x_all.at[pl.ds(my, 1)],
                     A1R, right)
        a1l = remote(x_all.at[pl.ds(my, 1)], x_all.at[pl.ds(my, 1)],
                     A1L, left)

        def issue_stage(t):
            j, qb = divmod(t, NQB)
            bb = batch_of(j)
            slot = t % 2
            ck = pltpu.make_async_copy(
                k_hbm.at[bb, :, qb, :, pl.ds(h0, HL), :], kst.at[slot],
                k_sems.at[slot])
            cv = pltpu.make_async_copy(
                v_hbm.at[bb, :, qb, :, pl.ds(h0, HL), :], vst.at[slot],
                v_sems.at[slot])
            ck.start()
            cv.start()
            return (ck, cv)

        desc = {} if _SKIP_DMA else {0: issue_stage(0), 1: issue_stage(1)}

        wq_bf[...] = wq_ref[...].astype(jnp.bfloat16)
        wo_bf[...] = wo_ref[...].astype(jnp.bfloat16)

        def compute_batch(j, store):
            bb = batch_of(j)
            xb = x_all[pl.ds(bb, 1)][0]
            q = jnp.dot(xb, wq_bf[...],
                        preferred_element_type=jnp.float32)
            q_bf[...] = (q * SCALE).astype(jnp.bfloat16)

            for qb in range(NQB):
                t = j * NQB + qb
                slot = t % 2
                if not _SKIP_DMA:
                    ck, cv = desc.pop(t)
                    ck.wait()
                    cv.wait()
                    k_bf[...] = kst[slot].reshape(KSEL, HL * DH
                                                  ).astype(jnp.bfloat16)
                    v_bf[...] = vst[slot].reshape(KSEL, HL * DH
                                                  ).astype(jnp.bfloat16)
                    if t + 2 < N_DEV * NQB:
                        desc[t + 2] = issue_stage(t + 2)

                if not _SKIP_MATH:
                    for h in range(HL):
                        kh = k_bf[:, h * DH:(h + 1) * DH]
                        qh = q_bf[qb * QBLK:(qb + 1) * QBLK,
                                  h * DH:(h + 1) * DH]
                        scores[h] = lax.dot_general(
                            qh, kh, (((1,), (1,)), ((), ())),
                            preferred_element_type=jnp.float32)
                    e = jnp.exp(scores[...])
                    w = e / jnp.sum(e, axis=-1, keepdims=True)
                    e_buf[...] = w.astype(jnp.bfloat16)
                    for h in range(HL):
                        o = jnp.dot(e_buf[h], v_bf[:, h * DH:(h + 1) * DH],
                                    preferred_element_type=jnp.float32)
                        ctx_bf[:, h * DH:(h + 1) * DH] = o.astype(
                            jnp.bfloat16)
                psum = jnp.dot(ctx_bf[...], wo_bf[...],
                               preferred_element_type=jnp.float32)
                store(qb, psum)

        def store_f32(ref):
            def f(qb, psum):
                ref[0, qb * QBLK:(qb + 1) * QBLK, :] = psum
            return f

        def store_bf16(ref):
            def f(qb, psum):
                ref[0, qb * QBLK:(qb + 1) * QBLK, :] = psum.astype(
                    jnp.bfloat16)
            return f

        compute_batch(0, store_f32(acc_my))

        a1r.wait_recv()
        a1l.wait_recv()
        a2r = remote(x_all.at[pl.ds(left, 1), pl.ds(0, HALF), :],
                     x_all.at[pl.ds(left, 1), pl.ds(0, HALF), :],
                     A2R, right)
        a2l = remote(x_all.at[pl.ds(right, 1), pl.ds(HALF, HALF), :],
                     x_all.at[pl.ds(right, 1), pl.ds(HALF, HALF), :],
                     A2L, left)

        compute_batch(1, store_bf16(rs_out_r))
        t1r = remote(rs_out_r.at[...], rs_in_a.at[...], T1R, right)

        a2r.wait_recv()
        a2l.wait_recv()
        compute_batch(2, store_bf16(rs_diag))

        t2r = remote(rs_diag.at[:, pl.ds(0, HALF), :], relay_r.at[...],
                     T2R, right)
        t2l = remote(rs_diag.at[:, pl.ds(HALF, HALF), :], relay_l.at[...],
                     T2L, left)

        compute_batch(3, store_bf16(rs_out_l))
        t1l = remote(rs_out_l.at[...], rs_in_b.at[...], T1L, left)

        t2r.wait_recv()
        t3r = remote(relay_r.at[...], diag_a.at[...], T3R, right)
        t2l.wait_recv()
        t3l = remote(relay_l.at[...], diag_b.at[...], T3L, left)

        t1r.wait_recv()
        t1l.wait_recv()
        base = (acc_my[...] + rs_in_a[...].astype(jnp.float32)
                + rs_in_b[...].astype(jnp.float32))

        t3r.wait_recv()
        t3l.wait_recv()
        out_ref[:, 0:HALF, :] = (base[:, 0:HALF, :]
                                 + diag_a[...].astype(jnp.float32))
        out_ref[:, HALF:SQ, :] = (base[:, HALF:SQ, :]
                                  + diag_b[...].astype(jnp.float32))

        for r in (a1r, a1l, a2r, a2l, t1r, t1l, t2r, t2l, t3r, t3l):
            r.wait_send()

    return pl.pallas_call(
        body,
        out_shape=jax.ShapeDtypeStruct((1, SQ, DM), jnp.float32),
        in_specs=[
            pl.BlockSpec(memory_space=pltpu.MemorySpace.VMEM),
            pl.BlockSpec(memory_space=pltpu.MemorySpace.VMEM),
            pl.BlockSpec(memory_space=pl.ANY),
            pl.BlockSpec(memory_space=pl.ANY),
            pl.BlockSpec(memory_space=pltpu.MemorySpace.VMEM),
        ],
        out_specs=pl.BlockSpec(memory_space=pltpu.MemorySpace.VMEM),
        scratch_shapes=[
            pltpu.VMEM((B, SQ, DM), jnp.bfloat16),
            pltpu.VMEM((DM, DM), jnp.bfloat16),
            pltpu.VMEM((DM, DM), jnp.bfloat16),
            pltpu.VMEM((SQ, HL * DH), jnp.bfloat16),
            pltpu.VMEM((2, NT, QBLK, HL, DH), jnp.float32),
            pltpu.VMEM((2, NT, QBLK, HL, DH), jnp.float32),
            pltpu.VMEM((KSEL, HL * DH), jnp.bfloat16),
            pltpu.VMEM((KSEL, HL * DH), jnp.bfloat16),
            pltpu.VMEM((QBLK, HL * DH), jnp.bfloat16),
            pltpu.VMEM((HL, QBLK, KSEL), jnp.float32),
            pltpu.VMEM((HL, QBLK, KSEL), jnp.bfloat16),
            pltpu.VMEM((1, SQ, DM), jnp.float32),
            pltpu.VMEM((1, SQ, DM), jnp.bfloat16),
            pltpu.VMEM((1, SQ, DM), jnp.bfloat16),
            pltpu.VMEM((1, SQ, DM), jnp.bfloat16),
            pltpu.VMEM((1, SQ, DM), jnp.bfloat16),
            pltpu.VMEM((1, SQ, DM), jnp.bfloat16),
            pltpu.VMEM((1, HALF, DM), jnp.bfloat16),
            pltpu.VMEM((1, HALF, DM), jnp.bfloat16),
            pltpu.VMEM((1, HALF, DM), jnp.bfloat16),
            pltpu.VMEM((1, HALF, DM), jnp.bfloat16),
            pltpu.SemaphoreType.DMA((10,)),
            pltpu.SemaphoreType.DMA((10,)),
            pltpu.SemaphoreType.DMA((2,)),
            pltpu.SemaphoreType.DMA((2,)),
        ],
        compiler_params=pltpu.CompilerParams(
            collective_id=0, vmem_limit_bytes=64 * 1024 * 1024),
    )(x, Wq, K_r, V_r, Wo)


# device time: 80340 ns/iter; 1.0907x vs baseline; 1.0197x over previous
import os

import jax
import jax.numpy as jnp
from jax import lax
from jax.experimental import pallas as pl
from jax.experimental.pallas import tpu as pltpu

N_DEV = 4
B, SQ, DM = 4, 256, 1024
HG, HL, DH = 32, 8, 128
NQB, QBLK = 4, 64
NT = 16
KSEL = NT * QBLK
HALF = SQ // 2
SCALE = 0.08838834764831843
SCALE2 = SCALE * 1.4426950408889634
ABLATE = os.environ.get("ABLATE", "")
_SKIP_DMA = ABLATE == "compute"
_SKIP_MATH = ABLATE == "mem"
OFF = (0, 1, 2, 3)

A1R, A1L, A2R, A2L, T1R, T1L, T2R, T2L, T3R, T3L = range(10)


def kernel(x, Wq, K_ext, V_ext, Wo):
    K_r = K_ext.reshape(B, NT, NQB, QBLK, HG, DH)
    V_r = V_ext.reshape(B, NT, NQB, QBLK, HG, DH)

    def body(x_ref, wq_ref, k_hbm, v_hbm, wo_ref, out_ref,
             x_all, wq_bf, wo_bf, q_bf, kst, vst, k_bf, v_bf, ctx_bf,
             scores, e_buf,
             acc_my, rs_out_r, rs_out_l, rs_diag, rs_in_a, rs_in_b,
             relay_r, relay_l, diag_a, diag_b,
             snd, rcv, k_sems, v_sems):
        my = lax.axis_index("i")
        right = lax.rem(my + 1, N_DEV)
        left = lax.rem(my + N_DEV - 1, N_DEV)
        h0 = my * HL

        def batch_of(j):
            return lax.rem(my + OFF[j], N_DEV)

        def remote(src, dst, i, dev):
            r = pltpu.make_async_remote_copy(
                src_ref=src, dst_ref=dst, send_sem=snd.at[i],
                recv_sem=rcv.at[i], device_id=(dev,),
                device_id_type=pl.DeviceIdType.MESH)
            r.start()
            return r

        barrier = pltpu.get_barrier_semaphore()
        for nbr in (left, right):
            pl.semaphore_signal(barrier, inc=1, device_id=(nbr,),
                                device_id_type=pl.DeviceIdType.MESH)
        pl.semaphore_wait(barrier, 2)

        x_all[pl.ds(my, 1)] = x_ref[...].astype(jnp.bfloat16)
        a1r = remote(x_all.at[pl.ds(my, 1)], x_all.at[pl.ds(my, 1)],
                     A1R, right)
        a1l = remote(x_all.at[pl.ds(my, 1)], x_all.at[pl.ds(my, 1)],
                     A1L, left)

        def issue_stage(t):
            j, qb = divmod(t, NQB)
            bb = batch_of(j)
            slot = t % 2
            ck = pltpu.make_async_copy(
                k_hbm.at[bb, :, qb, :, pl.ds(h0, HL), :], kst.at[slot],
                k_sems.at[slot])
            cv = pltpu.make_async_copy(
                v_hbm.at[bb, :, qb, :, pl.ds(h0, HL), :], vst.at[slot],
                v_sems.at[slot])
            ck.start()
            cv.start()
            return (ck, cv)

        desc = {} if _SKIP_DMA else {0: issue_stage(0), 1: issue_stage(1)}

        wq_bf[...] = wq_ref[...].astype(jnp.bfloat16)
        wo_bf[...] = wo_ref[...].astype(jnp.bfloat16)

        def compute_batch(j, store):
            bb = batch_of(j)
            xb = x_all[pl.ds(bb, 1)][0]
            q = jnp.dot(xb, wq_bf[...],
                        preferred_element_type=jnp.float32)
            q_bf[...] = (q * SCALE2).astype(jnp.bfloat16)

            for qb in range(NQB):
                t = j * NQB + qb
                slot = t % 2
                if not _SKIP_DMA:
                    ck, cv = desc.pop(t)
                    ck.wait()
                    cv.wait()
                    k_bf[...] = kst[slot].reshape(KSEL, HL * DH
                                                  ).astype(jnp.bfloat16)
                    v_bf[...] = vst[slot].reshape(KSEL, HL * DH
                                                  ).astype(jnp.bfloat16)
                    if t + 2 < N_DEV * NQB:
                        desc[t + 2] = issue_stage(t + 2)

                if not _SKIP_MATH:
                    for h in range(HL):
                        kh = k_bf[:, h * DH:(h + 1) * DH]
                        qh = q_bf[qb * QBLK:(qb + 1) * QBLK,
                                  h * DH:(h + 1) * DH]
                        scores[h] = lax.dot_general(
                            qh, kh, (((1,), (1,)), ((), ())),
                            preferred_element_type=jnp.float32)
                    e = jnp.exp2(scores[...])
                    inv = 1.0 / jnp.sum(e, axis=-1, keepdims=True)
                    e_buf[...] = e.astype(jnp.bfloat16)
                    for h in range(HL):
                        o = jnp.dot(e_buf[h], v_bf[:, h * DH:(h + 1) * DH],
                                    preferred_element_type=jnp.float32)
                        ctx_bf[:, h * DH:(h + 1) * DH] = (o * inv[h]
                                                          ).astype(
                            jnp.bfloat16)
                psum = jnp.dot(ctx_bf[...], wo_bf[...],
                               preferred_element_type=jnp.float32)
                store(qb, psum)

        def store_f32(ref):
            def f(qb, psum):
                ref[0, qb * QBLK:(qb + 1) * QBLK, :] = psum
            return f

        def store_bf16(ref):
            def f(qb, psum):
                ref[0, qb * QBLK:(qb + 1) * QBLK, :] = psum.astype(
                    jnp.bfloat16)
            return f

        compute_batch(0, store_f32(acc_my))

        a1r.wait_recv()
        a1l.wait_recv()
        a2r = remote(x_all.at[pl.ds(left, 1), pl.ds(0, HALF), :],
                     x_all.at[pl.ds(left, 1), pl.ds(0, HALF), :],
                     A2R, right)
        a2l = remote(x_all.at[pl.ds(right, 1), pl.ds(HALF, HALF), :],
                     x_all.at[pl.ds(right, 1), pl.ds(HALF, HALF), :],
                     A2L, left)

        compute_batch(1, store_bf16(rs_out_r))
        t1r = remote(rs_out_r.at[...], rs_in_a.at[...], T1R, right)

        a2r.wait_recv()
        a2l.wait_recv()
        compute_batch(2, store_bf16(rs_diag))

        t2r = remote(rs_diag.at[:, pl.ds(0, HALF), :], relay_r.at[...],
                     T2R, right)
        t2l = remote(rs_diag.at[:, pl.ds(HALF, HALF), :], relay_l.at[...],
                     T2L, left)

        compute_batch(3, store_bf16(rs_out_l))
        t1l = remote(rs_out_l.at[...], rs_in_b.at[...], T1L, left)

        t2r.wait_recv()
        t3r = remote(relay_r.at[...], diag_a.at[...], T3R, right)
        t2l.wait_recv()
        t3l = remote(relay_l.at[...], diag_b.at[...], T3L, left)

        t1r.wait_recv()
        t1l.wait_recv()
        base = (acc_my[...] + rs_in_a[...].astype(jnp.float32)
                + rs_in_b[...].astype(jnp.float32))

        t3r.wait_recv()
        t3l.wait_recv()
        out_ref[:, 0:HALF, :] = (base[:, 0:HALF, :]
                                 + diag_a[...].astype(jnp.float32))
        out_ref[:, HALF:SQ, :] = (base[:, HALF:SQ, :]
                                  + diag_b[...].astype(jnp.float32))

        for r in (a1r, a1l, a2r, a2l, t1r, t1l, t2r, t2l, t3r, t3l):
            r.wait_send()

    return pl.pallas_call(
        body,
        out_shape=jax.ShapeDtypeStruct((1, SQ, DM), jnp.float32),
        in_specs=[
            pl.BlockSpec(memory_space=pltpu.MemorySpace.VMEM),
            pl.BlockSpec(memory_space=pltpu.MemorySpace.VMEM),
            pl.BlockSpec(memory_space=pl.ANY),
            pl.BlockSpec(memory_space=pl.ANY),
            pl.BlockSpec(memory_space=pltpu.MemorySpace.VMEM),
        ],
        out_specs=pl.BlockSpec(memory_space=pltpu.MemorySpace.VMEM),
        scratch_shapes=[
            pltpu.VMEM((B, SQ, DM), jnp.bfloat16),
            pltpu.VMEM((DM, DM), jnp.bfloat16),
            pltpu.VMEM((DM, DM), jnp.bfloat16),
            pltpu.VMEM((SQ, HL * DH), jnp.bfloat16),
            pltpu.VMEM((2, NT, QBLK, HL, DH), jnp.float32),
            pltpu.VMEM((2, NT, QBLK, HL, DH), jnp.float32),
            pltpu.VMEM((KSEL, HL * DH), jnp.bfloat16),
            pltpu.VMEM((KSEL, HL * DH), jnp.bfloat16),
            pltpu.VMEM((QBLK, HL * DH), jnp.bfloat16),
            pltpu.VMEM((HL, QBLK, KSEL), jnp.float32),
            pltpu.VMEM((HL, QBLK, KSEL), jnp.bfloat16),
            pltpu.VMEM((1, SQ, DM), jnp.float32),
            pltpu.VMEM((1, SQ, DM), jnp.bfloat16),
            pltpu.VMEM((1, SQ, DM), jnp.bfloat16),
            pltpu.VMEM((1, SQ, DM), jnp.bfloat16),
            pltpu.VMEM((1, SQ, DM), jnp.bfloat16),
            pltpu.VMEM((1, SQ, DM), jnp.bfloat16),
            pltpu.VMEM((1, HALF, DM), jnp.bfloat16),
            pltpu.VMEM((1, HALF, DM), jnp.bfloat16),
            pltpu.VMEM((1, HALF, DM), jnp.bfloat16),
            pltpu.VMEM((1, HALF, DM), jnp.bfloat16),
            pltpu.SemaphoreType.DMA((10,)),
            pltpu.SemaphoreType.DMA((10,)),
            pltpu.SemaphoreType.DMA((2,)),
            pltpu.SemaphoreType.DMA((2,)),
        ],
        compiler_params=pltpu.CompilerParams(
            collective_id=0, vmem_limit_bytes=64 * 1024 * 1024),
    )(x, Wq, K_r, V_r, Wo)
